# baseline (speedup 1.0000x reference)
import functools

import jax
import jax.numpy as jnp
from jax import lax
from jax.experimental import pallas as pl
from jax.experimental.pallas import tpu as pltpu
from jax.experimental.pallas import tpu_sc as plsc

_NC = 2
_NS = 16
_NW = _NC * _NS
_LANES = 16
_BC = 128
_DPAD = 200


def _sc_embed(packed, *, n_comp, n_lev, L, B):
    D = 1 + n_comp + n_lev
    per_tile_in = L * _BC
    groups = _BC // _LANES
    mesh = plsc.VectorSubcoreMesh(core_axis_name="c", subcore_axis_name="s",
                                  num_cores=_NC, num_subcores=_NS)

    @functools.partial(
        pl.kernel,
        out_type=jax.ShapeDtypeStruct((L, D, B), jnp.float32),
        mesh=mesh,
        scratch_types=[
            pltpu.VMEM((1, _DPAD, _BC), jnp.float32),
            pltpu.VMEM((1, _DPAD, _BC), jnp.float32),
            pltpu.VMEM((1, _DPAD, _BC), jnp.float32),
            pltpu.VMEM((1, _DPAD, _BC), jnp.float32),
            pltpu.VMEM((3 * per_tile_in,), jnp.int32),
            pltpu.SemaphoreType.DMA,
            pltpu.SemaphoreType.DMA,
            pltpu.SemaphoreType.DMA,
            pltpu.SemaphoreType.DMA,
        ],
        compiler_params=pltpu.CompilerParams(needs_layout_passes=False),
    )
    def body(in_hbm, out_hbm, stage0, stage1, stage2, stage3, in_v,
             sem0, sem1, sem2, sem3):
        wid = lax.axis_index("s") * _NC + lax.axis_index("c")
        b0 = pl.multiple_of(wid * _BC, _BC)
        ones = jnp.full((_LANES,), 1.0, jnp.float32)
        zeros = jnp.zeros((_LANES,), jnp.float32)
        zrow = jnp.zeros((_LANES,), jnp.int32)
        lane = lax.iota(jnp.int32, _LANES)
        stages = (stage0, stage1, stage2, stage3)
        sems = (sem0, sem1, sem2, sem3)

        in0 = wid * (3 * per_tile_in)
        cp_in = pltpu.async_copy(in_hbm.at[pl.ds(in0, 3 * per_tile_in)],
                                 in_v, sem0)

        def zero_flat(i, carry):
            r = i * _LANES + lane
            ch = r // _BC
            bi = r - ch * _BC
            for st in stages:
                plsc.store_scatter(st, [zrow, ch, bi], zeros)
            return carry

        lax.fori_loop(0, (_DPAD * _BC) // _LANES, zero_flat, 0)
        cp_in.wait()

        def scatter(stage, l, value_t):
            row_in = l * _BC

            def group(g, carry):
                o = row_in + g * _LANES
                c16 = in_v[pl.ds(o, _LANES)]
                v16 = in_v[pl.ds(per_tile_in + o, _LANES)]
                b16 = lane + g * _LANES
                if value_t:
                    t16 = plsc.bitcast(
                        in_v[pl.ds(2 * per_tile_in + o, _LANES)], jnp.float32)
                    plsc.store_scatter(stage, [zrow, zrow, b16], t16)
                val = ones if value_t else zeros
                plsc.store_scatter(stage, [zrow, 1 + c16, b16], val,
                                   mask=c16 < n_comp)
                plsc.store_scatter(stage, [zrow, (1 + n_comp) + v16, b16],
                                   val, mask=v16 < n_lev)
                return carry

            lax.fori_loop(0, groups, group, 0)

        def start(stage, sem, l):
            return pltpu.async_copy(
                stage.at[:, pl.ds(0, D), :],
                out_hbm.at[pl.ds(l, 1), :, pl.ds(b0, _BC)], sem)

        def wait(stage, sem, l):
            pltpu.make_async_copy(
                stage.at[:, pl.ds(0, D), :],
                out_hbm.at[pl.ds(l, 1), :, pl.ds(b0, _BC)], sem).wait()

        for l in range(4):
            scatter(stages[l], l, True)
            start(stages[l], sems[l], l)

        for l in range(4, L):
            s = l % 4
            wait(stages[s], sems[s], l - 4)
            scatter(stages[s], l - 4, False)
            scatter(stages[s], l, True)
            start(stages[s], sems[s], l)

        for l in range(L - 4, L):
            s = l % 4
            wait(stages[s], sems[s], l)

    return body(packed)


def kernel(components, levels, time_elapsed, comp_table, level_table):
    n_comp = comp_table.shape[1]
    n_lev = level_table.shape[1]
    B, L = components.shape

    def _arrange(x):
        return x.T.reshape(L, _NW, _BC).transpose(1, 0, 2)

    packed = jnp.stack([
        _arrange(components.astype(jnp.int32)),
        _arrange(levels.astype(jnp.int32)),
        _arrange(lax.bitcast_convert_type(time_elapsed, jnp.int32)),
    ], axis=1).reshape(-1)

    assert L % 2 == 0 and B % (_NW * _BC) == 0

    out_t = _sc_embed(packed, n_comp=n_comp, n_lev=n_lev, L=L, B=B)
    return out_t.transpose(2, 0, 1)


# speedup vs baseline: 1.0462x; 1.0462x over previous
import functools

import jax
import jax.numpy as jnp
from jax import lax
from jax.experimental import pallas as pl
from jax.experimental.pallas import tpu as pltpu
from jax.experimental.pallas import tpu_sc as plsc

_NC = 2
_NS = 16
_NW = _NC * _NS
_LANES = 16
_BC = 128
_DPAD = 200


def _sc_embed(packed, *, n_comp, n_lev, L, B):
    D = 1 + n_comp + n_lev
    per_tile_in = L * _BC
    groups = _BC // _LANES
    mesh = plsc.VectorSubcoreMesh(core_axis_name="c", subcore_axis_name="s",
                                  num_cores=_NC, num_subcores=_NS)

    @functools.partial(
        pl.kernel,
        out_type=jax.ShapeDtypeStruct((L, D, B), jnp.float32),
        mesh=mesh,
        scratch_types=[
            pltpu.VMEM((1, _DPAD, _BC), jnp.float32),
            pltpu.VMEM((1, _DPAD, _BC), jnp.float32),
            pltpu.VMEM((3 * per_tile_in,), jnp.int32),
            pltpu.SemaphoreType.DMA,
            pltpu.SemaphoreType.DMA,
        ],
        compiler_params=pltpu.CompilerParams(needs_layout_passes=False),
    )
    def body(in_hbm, out_hbm, stage0, stage1, in_v, sem0, sem1):
        wid = lax.axis_index("s") * _NC + lax.axis_index("c")
        b0 = pl.multiple_of(wid * _BC, _BC)
        ones = jnp.full((_LANES,), 1.0, jnp.float32)
        zeros = jnp.zeros((_LANES,), jnp.float32)
        zrow = jnp.zeros((_LANES,), jnp.int32)
        lane = lax.iota(jnp.int32, _LANES)
        stages = (stage0, stage1)
        sems = (sem0, sem1)

        in0 = wid * (3 * per_tile_in)
        cp_in = pltpu.async_copy(in_hbm.at[pl.ds(in0, 3 * per_tile_in)],
                                 in_v, sem0)

        def zero_flat(i, carry):
            r = i * _LANES + lane
            ch = r // _BC
            bi = r - ch * _BC
            plsc.store_scatter(stage0, [zrow, ch, bi], zeros)
            plsc.store_scatter(stage1, [zrow, ch, bi], zeros)
            return carry

        lax.fori_loop(0, (_DPAD * _BC) // _LANES, zero_flat, 0)
        cp_in.wait()

        def scatter(stage, l, value_t):
            row_in = l * _BC

            def group(g, carry):
                o = row_in + g * _LANES
                c16 = in_v[pl.ds(o, _LANES)]
                v16 = in_v[pl.ds(per_tile_in + o, _LANES)]
                b16 = lane + g * _LANES
                if value_t:
                    t16 = plsc.bitcast(
                        in_v[pl.ds(2 * per_tile_in + o, _LANES)], jnp.float32)
                    plsc.store_scatter(stage, [zrow, zrow, b16], t16)
                val = ones if value_t else zeros
                plsc.store_scatter(stage, [zrow, 1 + c16, b16], val,
                                   mask=c16 < n_comp)
                plsc.store_scatter(stage, [zrow, (1 + n_comp) + v16, b16],
                                   val, mask=v16 < n_lev)
                return carry

            lax.fori_loop(0, groups, group, 0)

        def start(stage, sem, l):
            return pltpu.async_copy(
                stage.at[:, pl.ds(0, D), :],
                out_hbm.at[pl.ds(l, 1), :, pl.ds(b0, _BC)], sem)

        def wait(stage, sem, l):
            pltpu.make_async_copy(
                stage.at[:, pl.ds(0, D), :],
                out_hbm.at[pl.ds(l, 1), :, pl.ds(b0, _BC)], sem).wait()

        for li in range(2):
            scatter(stages[li], li, True)
            start(stages[li], sems[li], li)

        def pair(cc, carry):
            for li in range(2):
                l = cc * 2 + li
                wait(stages[li], sems[li], l - 2)
                scatter(stages[li], l - 2, False)
                scatter(stages[li], l, True)
                start(stages[li], sems[li], l)
            return carry

        lax.fori_loop(1, L // 2, pair, 0)

        for li in range(2):
            wait(stages[li], sems[li], L - 2 + li)

    return body(packed)


def kernel(components, levels, time_elapsed, comp_table, level_table):
    n_comp = comp_table.shape[1]
    n_lev = level_table.shape[1]
    B, L = components.shape

    def _arrange(x):
        return x.T.reshape(L, _NW, _BC).transpose(1, 0, 2)

    packed = jnp.stack([
        _arrange(components.astype(jnp.int32)),
        _arrange(levels.astype(jnp.int32)),
        _arrange(lax.bitcast_convert_type(time_elapsed, jnp.int32)),
    ], axis=1).reshape(-1)

    assert L % 2 == 0 and B % (_NW * _BC) == 0

    out_t = _sc_embed(packed, n_comp=n_comp, n_lev=n_lev, L=L, B=B)
    return out_t.transpose(2, 0, 1)
